# baseline (device time: 90403 ns/iter reference)
import jax
import jax.numpy as jnp
from jax import lax
from jax.experimental import pallas as pl
from jax.experimental.pallas import tpu as pltpu

N_DEV = 4
NSEM = 20
NWB = 14


def kernel(x):
    m, n = x.shape
    hm = m // 2
    mq = m // 4
    me = m // 8
    qme = m // 16
    f32 = jnp.float32
    bf16 = jnp.bfloat16

    def body(x_ref, dummy_ref, out_ref, xv_ref, sbuf_ref, qrecv_ref,
             erecv_ref, ov_ref, gbuf_ref, send_sems, recv_sems, fetch_sems,
             wb_sems):
        del dummy_ref
        my = lax.axis_index("i")
        px = my // 2
        py = jnp.bitwise_and(jnp.bitwise_xor(my, px), 1)
        p_flip_y = jnp.bitwise_xor(my, 1)
        p_flip_x = 3 - my

        H = []
        for h, (P, qi, ei) in enumerate([
            ((p_flip_y, p_flip_x, p_flip_x, p_flip_y), py, px),
            ((p_flip_x, p_flip_y, p_flip_y, p_flip_x), px, py),
        ]):
            hb = h * hm
            H.append((h, hb, P, qi, ei, hb + qi * mq, hb + qi * mq + ei * me))

        fetch = {}
        for h, hb, P, qi, ei, qb, eb in H:
            sq = hb + (1 - qi) * mq
            for k, r in enumerate(
                (sq + (1 - ei) * me, sq + ei * me, qb + (1 - ei) * me, eb)
            ):
                cp = pltpu.make_async_copy(
                    x_ref.at[pl.ds(r, me), :],
                    xv_ref.at[pl.ds(r, me), :],
                    fetch_sems.at[4 * h + k],
                )
                cp.start()
                fetch[(h, k)] = (cp, r)

        barrier_sem = pltpu.get_barrier_semaphore()
        for nbr in (p_flip_y, p_flip_x):
            pl.semaphore_signal(
                barrier_sem, inc=1,
                device_id=(nbr,), device_id_type=pl.DeviceIdType.MESH,
            )

        def xchg(src, dst, dev, h, s):
            return pltpu.make_async_remote_copy(
                src_ref=src, dst_ref=dst,
                send_sem=send_sems.at[10 * h + s],
                recv_sem=recv_sems.at[10 * h + s],
                device_id=(dev,), device_id_type=pl.DeviceIdType.MESH,
            )

        for h, hb, P, qi, ei, qb, eb in H:
            cp, r = fetch[(h, 0)]
            cp.wait()
            sbuf_ref[h, pl.ds((1 - ei) * me, me), :] = xv_ref[
                pl.ds(r, me), :
            ].astype(bf16)
        pl.semaphore_wait(barrier_sem, 2)

        descs = []
        d1a, d1b, d2, d3, d4a, d4b = {}, {}, {}, {}, {}, {}
        wb = {}

        for h, hb, P, qi, ei, qb, eb in H:
            d1a[h] = xchg(
                sbuf_ref.at[h, pl.ds((1 - ei) * me, me), :],
                qrecv_ref.at[h, pl.ds((1 - ei) * me, me), :],
                P[0], h, 0,
            )
            d1a[h].start()
            descs.append(d1a[h])
        for h, hb, P, qi, ei, qb, eb in H:
            cp, r = fetch[(h, 1)]
            cp.wait()
            sbuf_ref[h, pl.ds(ei * me, me), :] = xv_ref[
                pl.ds(r, me), :
            ].astype(bf16)
            d1b[h] = xchg(
                sbuf_ref.at[h, pl.ds(ei * me, me), :],
                qrecv_ref.at[h, pl.ds(ei * me, me), :],
                P[0], h, 1,
            )
            d1b[h].start()
            descs.append(d1b[h])

        for h, hb, P, qi, ei, qb, eb in H:
            fetch[(h, 2)][0].wait()
            d1a[h].wait_recv()
            fwd = pl.ds((1 - ei) * me, me)
            qrecv_ref[h, fwd, :] = (
                xv_ref[pl.ds(qb + (1 - ei) * me, me), :]
                + qrecv_ref[h, fwd, :].astype(f32)
            ).astype(bf16)
            for c in (0, 1):
                fc = pl.ds((1 - ei) * me + c * qme, qme)
                d2[(h, c)] = xchg(
                    qrecv_ref.at[h, fc, :],
                    erecv_ref.at[h, pl.ds(c * qme, qme), :],
                    P[1], h, 2 + c,
                )
                d2[(h, c)].start()
                descs.append(d2[(h, c)])

        for h, hb, P, qi, ei, qb, eb in H:
            fetch[(h, 3)][0].wait()
            d1b[h].wait_recv()
            ov_ref[h] = (
                xv_ref[pl.ds(eb, me), :]
                + qrecv_ref[h, pl.ds(ei * me, me), :].astype(f32)
            ).astype(bf16)

        for h, hb, P, qi, ei, qb, eb in H:
            own0 = qi * mq + ei * me
            for c in (0, 1):
                cc = pl.ds(c * qme, qme)
                d2[(h, c)].wait_recv()
                ov_ref[h, cc, :] = ov_ref[h, cc, :] + erecv_ref[h, cc, :]
                ownc = pl.ds(own0 + c * qme, qme)
                d3[(h, c)] = xchg(
                    ov_ref.at[h, cc, :], gbuf_ref.at[h, ownc, :],
                    P[2], h, 4 + c,
                )
                d3[(h, c)].start()
                descs.append(d3[(h, c)])
                d4a[(h, c)] = xchg(
                    ov_ref.at[h, cc, :], gbuf_ref.at[h, ownc, :],
                    P[3], h, 6 + c,
                )
                d4a[(h, c)].start()
                descs.append(d4a[(h, c)])
            wb[(h, 0)] = pltpu.make_async_copy(
                ov_ref.at[h], out_ref.at[pl.ds(eb, me), :], wb_sems.at[7 * h],
            )
            wb[(h, 0)].start()

        for h, hb, P, qi, ei, qb, eb in H:
            oth0 = qi * mq + (1 - ei) * me
            for c in (0, 1):
                d3[(h, c)].wait_recv()
                oc = pl.ds(oth0 + c * qme, qme)
                d4b[(h, c)] = xchg(
                    gbuf_ref.at[h, oc, :], gbuf_ref.at[h, oc, :],
                    P[3], h, 8 + c,
                )
                d4b[(h, c)].start()
                descs.append(d4b[(h, c)])
                wb[(h, 1, c)] = pltpu.make_async_copy(
                    gbuf_ref.at[h, oc, :],
                    out_ref.at[pl.ds(hb + oth0 + c * qme, qme), :],
                    wb_sems.at[7 * h + 1 + c],
                )
                wb[(h, 1, c)].start()

        for h, hb, P, qi, ei, qb, eb in H:
            oq = (1 - qi) * mq
            for c in (0, 1):
                d4a[(h, c)].wait_recv()
                r = oq + ei * me + c * qme
                wb[(h, 2, c)] = pltpu.make_async_copy(
                    gbuf_ref.at[h, pl.ds(r, qme), :],
                    out_ref.at[pl.ds(hb + r, qme), :],
                    wb_sems.at[7 * h + 3 + c],
                )
                wb[(h, 2, c)].start()
            for c in (0, 1):
                d4b[(h, c)].wait_recv()
                r = oq + (1 - ei) * me + c * qme
                wb[(h, 3, c)] = pltpu.make_async_copy(
                    gbuf_ref.at[h, pl.ds(r, qme), :],
                    out_ref.at[pl.ds(hb + r, qme), :],
                    wb_sems.at[7 * h + 5 + c],
                )
                wb[(h, 3, c)].start()

        for k in wb:
            wb[k].wait()
        for d in descs:
            d.wait_send()

    return pl.pallas_call(
        body,
        out_shape=jax.ShapeDtypeStruct((m, n), bf16),
        in_specs=[
            pl.BlockSpec(memory_space=pl.ANY),
            pl.BlockSpec(memory_space=pl.ANY),
        ],
        out_specs=pl.BlockSpec(memory_space=pl.ANY),
        input_output_aliases={1: 0},
        scratch_shapes=[
            pltpu.VMEM((m, n), jnp.float32),
            pltpu.VMEM((2, mq, n), bf16),
            pltpu.VMEM((2, mq, n), bf16),
            pltpu.VMEM((2, me, n), bf16),
            pltpu.VMEM((2, me, n), bf16),
            pltpu.VMEM((2, hm, n), bf16),
            pltpu.SemaphoreType.DMA((NSEM,)),
            pltpu.SemaphoreType.DMA((NSEM,)),
            pltpu.SemaphoreType.DMA((8,)),
            pltpu.SemaphoreType.DMA((NWB,)),
        ],
        compiler_params=pltpu.CompilerParams(
            collective_id=0,
            vmem_limit_bytes=100 * 1024 * 1024,
        ),
    )(x, jnp.full((m, n), 0, bf16) + x[:1, :1].astype(bf16))
